# Initial kernel scaffold; baseline (speedup 1.0000x reference)
import jax
import jax.numpy as jnp
from jax.experimental import pallas as pl


def kernel(xyz, x):
    raise NotImplementedError("write your pallas kernel here")


# speedup vs baseline: 11.5580x; 11.5580x over previous
import functools

import jax
import jax.numpy as jnp
from jax import lax
from jax.experimental import pallas as pl
from jax.experimental.pallas import tpu as pltpu
from jax.experimental.pallas import tpu_sc as plsc

B = 4
N = 16384
M = 1024
K = 32
C_FEAT = 64
NSUB = 128
NLANE = 128
QBLK = 128

_BIG = 1e10


def _fps_body(xyzt_ref, idx_ref, lct_ref, dist_ref):
    x0 = xyzt_ref[0]
    x1 = xyzt_ref[1]
    x2 = xyzt_ref[2]
    ii = (lax.broadcasted_iota(jnp.int32, (B, NSUB, NLANE), 1) * NLANE
          + lax.broadcasted_iota(jnp.int32, (B, NSUB, NLANE), 2))
    im = lax.broadcasted_iota(jnp.int32, (B, M), 1)
    dist_ref[...] = jnp.full((B, NSUB, NLANE), _BIG, jnp.float32)

    def _rmin(a):
        return jnp.min(jnp.min(a, axis=2, keepdims=True), axis=1, keepdims=True)

    def _rmax(a):
        return jnp.max(jnp.max(a, axis=2, keepdims=True), axis=1, keepdims=True)

    def _rsum(a):
        return jnp.sum(jnp.sum(a, axis=2, keepdims=True), axis=1, keepdims=True)

    def body(i, far):
        sel = im == i
        idx_ref[...] = jnp.where(sel, far[:, :, 0], idx_ref[...])
        onehot = ii == far
        zero = jnp.float32(0.0)
        cx = _rsum(jnp.where(onehot, x0, zero))
        cy = _rsum(jnp.where(onehot, x1, zero))
        cz = _rsum(jnp.where(onehot, x2, zero))
        lct_ref[0] = jnp.where(sel, cx[:, :, 0], lct_ref[0])
        lct_ref[1] = jnp.where(sel, cy[:, :, 0], lct_ref[1])
        lct_ref[2] = jnp.where(sel, cz[:, :, 0], lct_ref[2])
        dx = x0 - cx
        dy = x1 - cy
        dz = x2 - cz
        d = dx * dx + dy * dy + dz * dz
        dist = jnp.minimum(dist_ref[...], d)
        dist_ref[...] = dist
        m = _rmax(dist)
        far_new = _rmin(jnp.where(dist == m, ii, jnp.int32(N)))
        return far_new

    lax.fori_loop(0, M, body, jnp.zeros((B, 1, 1), jnp.int32))


def _run_fps(xyz):
    xyzt = xyz.transpose(2, 0, 1).reshape(3, B, NSUB, NLANE)
    return pl.pallas_call(
        _fps_body,
        out_shape=[
            jax.ShapeDtypeStruct((B, M), jnp.int32),
            jax.ShapeDtypeStruct((3, B, M), jnp.float32),
        ],
        scratch_shapes=[pltpu.VMEM((B, NSUB, NLANE), jnp.float32)],
    )(xyzt)


def _knn_body(lc_ref, xyzt_ref, idx_ref, dist_ref):
    q = lc_ref[0]
    qx = q[:, 0:1]
    qy = q[:, 1:2]
    qz = q[:, 2:3]
    px = xyzt_ref[0, 0:1, :]
    py = xyzt_ref[0, 1:2, :]
    pz = xyzt_ref[0, 2:3, :]
    def _b(v):
        return v.astype(jnp.bfloat16).astype(jnp.float32)

    dot = _b(qx) * _b(px) + _b(qy) * _b(py) + _b(qz) * _b(pz)
    d = jnp.float32(-2.0) * dot
    d = d + (qx * qx + qy * qy + qz * qz)
    d = d + (px * px + py * py + pz * pz)
    dist_ref[...] = d
    ii = lax.broadcasted_iota(jnp.int32, (QBLK, N), 1)
    ik = lax.broadcasted_iota(jnp.int32, (QBLK, K), 1)

    def body(j, _):
        dcur = dist_ref[...]
        m = jnp.min(dcur, axis=1, keepdims=True)
        sel = jnp.where(dcur == m, ii, jnp.int32(N))
        idxj = jnp.min(sel, axis=1, keepdims=True)
        idx_ref[0] = jnp.where(ik == j, idxj, idx_ref[0])
        dist_ref[...] = jnp.where(ii == idxj, _BIG, dcur)
        return 0

    lax.fori_loop(0, K, body, 0)


def _run_knn(lc_xyz, xyz):
    xyzt = xyz.transpose(0, 2, 1)
    return pl.pallas_call(
        _knn_body,
        grid=(B, M // QBLK),
        in_specs=[
            pl.BlockSpec((1, QBLK, 3), lambda b, s: (b, s, 0)),
            pl.BlockSpec((1, 3, N), lambda b, s: (b, 0, 0)),
        ],
        out_specs=pl.BlockSpec((1, QBLK, K), lambda b, s: (b, s, 0)),
        out_shape=jax.ShapeDtypeStruct((B, M, K), jnp.int32),
        scratch_shapes=[pltpu.VMEM((QBLK, N), jnp.float32)],
    )(lc_xyz, xyzt)


_NC = 2
_NS = 16
_NW = _NC * _NS
_CHUNK = 128
_LC_PER_W = (B * M) // _NW
_KNN_PER_W = (B * M * K) // _NW
_KNN_CHUNKS = _KNN_PER_W // _CHUNK


def _sc_gather_body(xf_hbm, xyzp_hbm, fps_hbm, knn_hbm,
                    lcx_out, kxyz_out, kx_out,
                    lci_v, lcr_v, ki_v, kxyzr_v, kxr_v, sem1, sem2):
    wid = lax.axis_index("s") * _NC + lax.axis_index("c")
    pltpu.sync_copy(fps_hbm.at[pl.ds(wid * _LC_PER_W, _LC_PER_W)], lci_v)
    pltpu.async_copy(xf_hbm.at[lci_v], lcr_v, sem1).wait()
    pltpu.sync_copy(lcr_v, lcx_out.at[pl.ds(wid * _LC_PER_W, _LC_PER_W)])
    pltpu.sync_copy(knn_hbm.at[pl.ds(wid * _KNN_CHUNKS, _KNN_CHUNKS)], ki_v)

    def chunk(j, _):
        row = ki_v.at[j]
        cp1 = pltpu.async_copy(xyzp_hbm.at[row], kxyzr_v, sem1)
        cp2 = pltpu.async_copy(xf_hbm.at[row], kxr_v, sem2)
        cp1.wait()
        cp2.wait()
        base = wid * _KNN_PER_W + j * _CHUNK
        pltpu.sync_copy(kxyzr_v, kxyz_out.at[pl.ds(base, _CHUNK)])
        pltpu.sync_copy(kxr_v, kx_out.at[pl.ds(base, _CHUNK)])
        return 0

    lax.fori_loop(0, _KNN_CHUNKS, chunk, 0)


@functools.cache
def _make_sc_gather():
    return functools.partial(
        pl.kernel,
        mesh=plsc.VectorSubcoreMesh(core_axis_name="c", subcore_axis_name="s",
                                    num_cores=_NC, num_subcores=_NS),
        out_type=[
            jax.ShapeDtypeStruct((B * M, C_FEAT), jnp.float32),
            jax.ShapeDtypeStruct((B * M * K, 16), jnp.float32),
            jax.ShapeDtypeStruct((B * M * K, C_FEAT), jnp.float32),
        ],
        scratch_types=[
            pltpu.VMEM((_LC_PER_W,), jnp.int32),
            pltpu.VMEM((_LC_PER_W, C_FEAT), jnp.float32),
            pltpu.VMEM((_KNN_CHUNKS, _CHUNK), jnp.int32),
            pltpu.VMEM((_CHUNK, 16), jnp.float32),
            pltpu.VMEM((_CHUNK, C_FEAT), jnp.float32),
            pltpu.SemaphoreType.DMA,
            pltpu.SemaphoreType.DMA,
        ],
        compiler_params=pltpu.CompilerParams(use_tc_tiling_on_sc=False),
    )(_sc_gather_body)


def kernel(xyz, x):
    fps_idx, lct = _run_fps(xyz)
    lc_xyz = lct.transpose(1, 2, 0)
    knn_idx = _run_knn(lc_xyz, xyz)

    xf = x.reshape(B * N, C_FEAT)
    xyzp = jnp.pad(xyz, ((0, 0), (0, 0), (0, 13))).reshape(B * N, 16)
    base = jnp.arange(B, dtype=jnp.int32) * N
    fps_flat = (fps_idx + base[:, None]).reshape(-1)
    knn_flat = (knn_idx + base[:, None, None]).reshape(B * M * K // _CHUNK,
                                                       _CHUNK)
    lcx_rows, kxyz_rows, kx_rows = _make_sc_gather()(xf, xyzp, fps_flat,
                                                     knn_flat)

    lc_x = lcx_rows.reshape(B, M, C_FEAT)
    knn_xyz = kxyz_rows.reshape(B, M, K, 16)[..., :3]
    knn_x = kx_rows.reshape(B, M, K, C_FEAT)
    return lc_xyz, lc_x, knn_xyz, knn_x


# speedup vs baseline: 13.3657x; 1.1564x over previous
import functools

import jax
import jax.numpy as jnp
from jax import lax
from jax.experimental import pallas as pl
from jax.experimental.pallas import tpu as pltpu
from jax.experimental.pallas import tpu_sc as plsc

B = 4
N = 16384
M = 1024
K = 32
C_FEAT = 64
NSUB = 128
NLANE = 128
QBLK = 128

_BIG = 1e10


def _fps_body(xyzt_ref, idx_ref, lct_ref, dist_ref):
    x0 = xyzt_ref[0]
    x1 = xyzt_ref[1]
    x2 = xyzt_ref[2]
    ii = (lax.broadcasted_iota(jnp.int32, (B, NSUB, NLANE), 1) * NLANE
          + lax.broadcasted_iota(jnp.int32, (B, NSUB, NLANE), 2))
    im = lax.broadcasted_iota(jnp.int32, (B, M), 1)
    dist_ref[...] = jnp.full((B, NSUB, NLANE), _BIG, jnp.float32)

    def _rmin(a):
        return jnp.min(jnp.min(a, axis=2, keepdims=True), axis=1, keepdims=True)

    def _rmax(a):
        return jnp.max(jnp.max(a, axis=2, keepdims=True), axis=1, keepdims=True)

    def _rsum(a):
        return jnp.sum(jnp.sum(a, axis=2, keepdims=True), axis=1, keepdims=True)

    def body(i, far):
        sel = im == i
        idx_ref[...] = jnp.where(sel, far[:, :, 0], idx_ref[...])
        onehot = ii == far
        zero = jnp.float32(0.0)
        cx = _rsum(jnp.where(onehot, x0, zero))
        cy = _rsum(jnp.where(onehot, x1, zero))
        cz = _rsum(jnp.where(onehot, x2, zero))
        lct_ref[0] = jnp.where(sel, cx[:, :, 0], lct_ref[0])
        lct_ref[1] = jnp.where(sel, cy[:, :, 0], lct_ref[1])
        lct_ref[2] = jnp.where(sel, cz[:, :, 0], lct_ref[2])
        dx = x0 - cx
        dy = x1 - cy
        dz = x2 - cz
        d = dx * dx + dy * dy + dz * dz
        dist = jnp.minimum(dist_ref[...], d)
        dist_ref[...] = dist
        m = _rmax(dist)
        far_new = _rmin(jnp.where(dist == m, ii, jnp.int32(N)))
        return far_new

    lax.fori_loop(0, M, body, jnp.zeros((B, 1, 1), jnp.int32))


def _run_fps(xyz):
    xyzt = xyz.transpose(2, 0, 1).reshape(3, B, NSUB, NLANE)
    return pl.pallas_call(
        _fps_body,
        out_shape=[
            jax.ShapeDtypeStruct((B, M), jnp.int32),
            jax.ShapeDtypeStruct((3, B, M), jnp.float32),
        ],
        scratch_shapes=[pltpu.VMEM((B, NSUB, NLANE), jnp.float32)],
    )(xyzt)


NCH = 128
NEH = 16
NEL = 8
TPC = 6
_SH4 = (QBLK, NEH, NEL, NCH)


def _knn_body(lc_ref, xyzr_ref, idx_ref, dist_ref, cv_ref, ci_ref):
    q = lc_ref[0]
    qx = q[:, 0:1].reshape(QBLK, 1, 1, 1)
    qy = q[:, 1:2].reshape(QBLK, 1, 1, 1)
    qz = q[:, 2:3].reshape(QBLK, 1, 1, 1)
    px = xyzr_ref[0:1]
    py = xyzr_ref[1:2]
    pz = xyzr_ref[2:3]

    def _b(v):
        return v.astype(jnp.bfloat16).astype(jnp.float32)

    dot = _b(qx) * _b(px) + _b(qy) * _b(py) + _b(qz) * _b(pz)
    d = jnp.float32(-2.0) * dot
    d = d + (qx * qx + qy * qy + qz * qz)
    d = d + (px * px + py * py + pz * pz)
    dist_ref[...] = d

    ic = lax.broadcasted_iota(jnp.int32, _SH4, 3)
    ieh = lax.broadcasted_iota(jnp.int32, _SH4, 1)
    iel = lax.broadcasted_iota(jnp.int32, _SH4, 2)
    ii4 = ic * (NEH * NEL) + ieh * NEL + iel
    ik = lax.broadcasted_iota(jnp.int32, (QBLK, K), 1)

    def _vmin(a):
        return jnp.min(jnp.min(a, axis=2, keepdims=True), axis=1,
                       keepdims=True)

    dcur = d
    mt = _vmin(d)
    for t in range(TPC):
        it = _vmin(jnp.where(dcur == mt, ii4, jnp.int32(N)))
        cv_ref[:, t, :] = mt[:, 0, 0, :]
        ci_ref[:, t, :] = it[:, 0, 0, :]
        if t + 1 < TPC:
            dcur = jnp.where(ii4 == it, _BIG, dcur)
            mt = _vmin(dcur)

    c6v = cv_ref[:, TPC - 1, :]
    c6i = ci_ref[:, TPC - 1, :]

    def body(j, _):
        cv = cv_ref[...]
        civ = ci_ref[...]
        m = jnp.min(jnp.min(cv, axis=2, keepdims=True), axis=1,
                    keepdims=True)
        sel = jnp.where(cv == m, civ, jnp.int32(N))
        idxj = jnp.min(jnp.min(sel, axis=2, keepdims=True), axis=1,
                       keepdims=True)
        idx_ref[0] = jnp.where(ik == j, idxj[:, :, 0], idx_ref[0])
        cv_ref[...] = jnp.where(civ == idxj, _BIG, cv)
        return (m[:, :, 0], idxj[:, :, 0])

    z2 = jnp.zeros((QBLK, 1), jnp.float32), jnp.zeros((QBLK, 1), jnp.int32)
    v32, i32v = lax.fori_loop(0, K, body, z2)

    v32b = v32.reshape(QBLK, 1)
    i32b = i32v.reshape(QBLK, 1)
    viol = (c6v < v32b) | ((c6v == v32b) & (c6i <= i32b))
    nviol = jnp.sum(jnp.sum(viol.astype(jnp.int32), axis=1, keepdims=True),
                    axis=0, keepdims=True)

    @pl.when(nviol[0, 0] > 0)
    def _fallback():
        def fbody(j, m):
            dc = dist_ref[...]
            sel = jnp.where(dc == m, ii4, jnp.int32(N))
            idxj = jnp.min(jnp.min(jnp.min(
                sel, axis=2, keepdims=True), axis=1, keepdims=True),
                axis=3, keepdims=True)
            idx_ref[0] = jnp.where(ik == j, idxj[:, :, 0, 0], idx_ref[0])
            dn = jnp.where(ii4 == idxj, _BIG, dc)
            dist_ref[...] = dn
            return jnp.min(_vmin(dn), axis=3, keepdims=True)

        m0 = jnp.min(_vmin(dist_ref[...]), axis=3, keepdims=True)
        lax.fori_loop(0, K, fbody, m0)


def _run_knn(lc_xyz, xyz):
    xyzr = xyz.reshape(B, NCH, NEH, NEL, 3).transpose(0, 4, 2, 3, 1)
    xyzr = xyzr.reshape(B * 3, NEH, NEL, NCH)
    return pl.pallas_call(
        _knn_body,
        grid=(B, M // QBLK),
        in_specs=[
            pl.BlockSpec((1, QBLK, 3), lambda b, s: (b, s, 0)),
            pl.BlockSpec((3, NEH, NEL, NCH), lambda b, s: (b, 0, 0, 0)),
        ],
        out_specs=pl.BlockSpec((1, QBLK, K), lambda b, s: (b, s, 0)),
        out_shape=jax.ShapeDtypeStruct((B, M, K), jnp.int32),
        scratch_shapes=[
            pltpu.VMEM(_SH4, jnp.float32),
            pltpu.VMEM((QBLK, TPC, NCH), jnp.float32),
            pltpu.VMEM((QBLK, TPC, NCH), jnp.int32),
        ],
    )(lc_xyz, xyzr)


_NC = 2
_NS = 16
_NW = _NC * _NS
_CHUNK = 128
_LC_PER_W = (B * M) // _NW
_KNN_PER_W = (B * M * K) // _NW
_KNN_CHUNKS = _KNN_PER_W // _CHUNK


def _sc_gather_body(xf_hbm, xyzp_hbm, fps_hbm, knn_hbm,
                    lcx_out, kxyz_out, kx_out,
                    lci_v, lcr_v, ki_v, kxyzr_v, kxr_v, sem1, sem2):
    wid = lax.axis_index("s") * _NC + lax.axis_index("c")
    pltpu.sync_copy(fps_hbm.at[pl.ds(wid * _LC_PER_W, _LC_PER_W)], lci_v)
    pltpu.async_copy(xf_hbm.at[lci_v], lcr_v, sem1).wait()
    pltpu.sync_copy(lcr_v, lcx_out.at[pl.ds(wid * _LC_PER_W, _LC_PER_W)])
    pltpu.sync_copy(knn_hbm.at[pl.ds(wid * _KNN_CHUNKS, _KNN_CHUNKS)], ki_v)

    def chunk(j, _):
        row = ki_v.at[j]
        cp1 = pltpu.async_copy(xyzp_hbm.at[row], kxyzr_v, sem1)
        cp2 = pltpu.async_copy(xf_hbm.at[row], kxr_v, sem2)
        cp1.wait()
        cp2.wait()
        base = wid * _KNN_PER_W + j * _CHUNK
        pltpu.sync_copy(kxyzr_v, kxyz_out.at[pl.ds(base, _CHUNK)])
        pltpu.sync_copy(kxr_v, kx_out.at[pl.ds(base, _CHUNK)])
        return 0

    lax.fori_loop(0, _KNN_CHUNKS, chunk, 0)


@functools.cache
def _make_sc_gather():
    return functools.partial(
        pl.kernel,
        mesh=plsc.VectorSubcoreMesh(core_axis_name="c", subcore_axis_name="s",
                                    num_cores=_NC, num_subcores=_NS),
        out_type=[
            jax.ShapeDtypeStruct((B * M, C_FEAT), jnp.float32),
            jax.ShapeDtypeStruct((B * M * K, 16), jnp.float32),
            jax.ShapeDtypeStruct((B * M * K, C_FEAT), jnp.float32),
        ],
        scratch_types=[
            pltpu.VMEM((_LC_PER_W,), jnp.int32),
            pltpu.VMEM((_LC_PER_W, C_FEAT), jnp.float32),
            pltpu.VMEM((_KNN_CHUNKS, _CHUNK), jnp.int32),
            pltpu.VMEM((_CHUNK, 16), jnp.float32),
            pltpu.VMEM((_CHUNK, C_FEAT), jnp.float32),
            pltpu.SemaphoreType.DMA,
            pltpu.SemaphoreType.DMA,
        ],
        compiler_params=pltpu.CompilerParams(use_tc_tiling_on_sc=False),
    )(_sc_gather_body)


def kernel(xyz, x):
    fps_idx, lct = _run_fps(xyz)
    lc_xyz = lct.transpose(1, 2, 0)
    knn_idx = _run_knn(lc_xyz, xyz)

    xf = x.reshape(B * N, C_FEAT)
    xyzp = jnp.pad(xyz, ((0, 0), (0, 0), (0, 13))).reshape(B * N, 16)
    base = jnp.arange(B, dtype=jnp.int32) * N
    fps_flat = (fps_idx + base[:, None]).reshape(-1)
    knn_flat = (knn_idx + base[:, None, None]).reshape(B * M * K // _CHUNK,
                                                       _CHUNK)
    lcx_rows, kxyz_rows, kx_rows = _make_sc_gather()(xf, xyzp, fps_flat,
                                                     knn_flat)

    lc_x = lcx_rows.reshape(B, M, C_FEAT)
    knn_xyz = kxyz_rows.reshape(B, M, K, 16)[..., :3]
    knn_x = kx_rows.reshape(B, M, K, C_FEAT)
    return lc_xyz, lc_x, knn_xyz, knn_x


# speedup vs baseline: 16.4798x; 1.2330x over previous
import functools

import jax
import jax.numpy as jnp
from jax import lax
from jax.experimental import pallas as pl
from jax.experimental.pallas import tpu as pltpu
from jax.experimental.pallas import tpu_sc as plsc

B = 4
N = 16384
M = 1024
K = 32
C_FEAT = 64
NSUB = 128
NLANE = 128
QBLK = 128

_BIG = 1e10


def _fps_body(xyzt_ref, idx_ref, lct_ref, dist_ref):
    x0 = xyzt_ref[0]
    x1 = xyzt_ref[1]
    x2 = xyzt_ref[2]
    ii = (lax.broadcasted_iota(jnp.int32, (B, NSUB, NLANE), 1) * NLANE
          + lax.broadcasted_iota(jnp.int32, (B, NSUB, NLANE), 2))
    im = lax.broadcasted_iota(jnp.int32, (B, M), 1)
    dist_ref[...] = jnp.full((B, NSUB, NLANE), _BIG, jnp.float32)

    def _rmin(a):
        return jnp.min(jnp.min(a, axis=2, keepdims=True), axis=1, keepdims=True)

    def _rmax(a):
        return jnp.max(jnp.max(a, axis=2, keepdims=True), axis=1, keepdims=True)

    def _rsum(a):
        return jnp.sum(jnp.sum(a, axis=2, keepdims=True), axis=1, keepdims=True)

    def body(i, far):
        sel = im == i
        idx_ref[...] = jnp.where(sel, far[:, :, 0], idx_ref[...])
        onehot = ii == far
        zero = jnp.float32(0.0)
        cx = _rsum(jnp.where(onehot, x0, zero))
        cy = _rsum(jnp.where(onehot, x1, zero))
        cz = _rsum(jnp.where(onehot, x2, zero))
        lct_ref[0] = jnp.where(sel, cx[:, :, 0], lct_ref[0])
        lct_ref[1] = jnp.where(sel, cy[:, :, 0], lct_ref[1])
        lct_ref[2] = jnp.where(sel, cz[:, :, 0], lct_ref[2])
        dx = x0 - cx
        dy = x1 - cy
        dz = x2 - cz
        d = dx * dx + dy * dy + dz * dz
        dist = jnp.minimum(dist_ref[...], d)
        dist_ref[...] = dist
        m = _rmax(dist)
        far_new = _rmin(jnp.where(dist == m, ii, jnp.int32(N)))
        return far_new

    lax.fori_loop(0, M, body, jnp.zeros((B, 1, 1), jnp.int32))


def _run_fps(xyz):
    xyzt = xyz.transpose(2, 0, 1).reshape(3, B, NSUB, NLANE)
    return pl.pallas_call(
        _fps_body,
        out_shape=[
            jax.ShapeDtypeStruct((B, M), jnp.int32),
            jax.ShapeDtypeStruct((3, B, M), jnp.float32),
        ],
        scratch_shapes=[pltpu.VMEM((B, NSUB, NLANE), jnp.float32)],
    )(xyzt)


NCH = 128
NEH = 16
NEL = 8
TPC = 6
_SH4 = (QBLK, NEH, NEL, NCH)


def _knn_body(lc_ref, xyzr_ref, idx_ref, dist_ref, cv_ref, ci_ref):
    q = lc_ref[0]
    qx = q[:, 0:1].reshape(QBLK, 1, 1, 1)
    qy = q[:, 1:2].reshape(QBLK, 1, 1, 1)
    qz = q[:, 2:3].reshape(QBLK, 1, 1, 1)
    px = xyzr_ref[0:1]
    py = xyzr_ref[1:2]
    pz = xyzr_ref[2:3]

    def _b(v):
        return v.astype(jnp.bfloat16).astype(jnp.float32)

    dot = _b(qx) * _b(px) + _b(qy) * _b(py) + _b(qz) * _b(pz)
    d = jnp.float32(-2.0) * dot
    d = d + (qx * qx + qy * qy + qz * qz)
    d = d + (px * px + py * py + pz * pz)
    dist_ref[...] = d

    sh3 = (QBLK, NEL, NCH)
    iel3 = lax.broadcasted_iota(jnp.int32, sh3, 1)
    ic3 = lax.broadcasted_iota(jnp.int32, sh3, 2)
    ik = lax.broadcasted_iota(jnp.int32, (QBLK, K), 1)
    iiN = jnp.int32(N)

    def _ii(eh):
        return ic3 * (NEH * NEL) + (eh * NEL) + iel3

    acc = dist_ref[:, 0]
    for eh in range(1, NEH):
        acc = jnp.minimum(acc, dist_ref[:, eh])
    mt = jnp.min(acc, axis=1, keepdims=True)
    for t in range(TPC):
        iacc = jnp.where(dist_ref[:, 0] == mt, _ii(0), iiN)
        for eh in range(1, NEH):
            iacc = jnp.minimum(
                iacc, jnp.where(dist_ref[:, eh] == mt, _ii(eh), iiN))
        it = jnp.min(iacc, axis=1, keepdims=True)
        cv_ref[:, t * NCH:(t + 1) * NCH] = mt[:, 0, :]
        ci_ref[:, t * NCH:(t + 1) * NCH] = it[:, 0, :]
        if t + 1 < TPC:
            dn = jnp.where(_ii(0) == it, _BIG, dist_ref[:, 0])
            dist_ref[:, 0] = dn
            acc = dn
            for eh in range(1, NEH):
                dn = jnp.where(_ii(eh) == it, _BIG, dist_ref[:, eh])
                dist_ref[:, eh] = dn
                acc = jnp.minimum(acc, dn)
            mt = jnp.min(acc, axis=1, keepdims=True)

    c6v = cv_ref[:, (TPC - 1) * NCH:TPC * NCH]
    c6i = ci_ref[:, (TPC - 1) * NCH:TPC * NCH]

    def body(j, _):
        cv = cv_ref[...]
        civ = ci_ref[...]
        m = jnp.min(cv, axis=1, keepdims=True)
        sel = jnp.where(cv == m, civ, iiN)
        idxj = jnp.min(sel, axis=1, keepdims=True)
        idx_ref[0] = jnp.where(ik == j, idxj, idx_ref[0])
        cv_ref[...] = jnp.where(civ == idxj, _BIG, cv)
        return (m, idxj)

    z2 = jnp.zeros((QBLK, 1), jnp.float32), jnp.zeros((QBLK, 1), jnp.int32)
    v32, i32v = lax.fori_loop(0, K, body, z2)

    viol = (c6v < v32) | ((c6v == v32) & (c6i <= i32v))
    nviol = jnp.sum(jnp.sum(viol.astype(jnp.int32), axis=1, keepdims=True),
                    axis=0, keepdims=True)

    @pl.when(nviol[0, 0] > 0)
    def _fallback():
        dist_ref[...] = d
        ic = lax.broadcasted_iota(jnp.int32, _SH4, 3)
        ieh = lax.broadcasted_iota(jnp.int32, _SH4, 1)
        iel = lax.broadcasted_iota(jnp.int32, _SH4, 2)
        ii4 = ic * (NEH * NEL) + ieh * NEL + iel

        def _vmin(a):
            return jnp.min(jnp.min(a, axis=2, keepdims=True), axis=1,
                           keepdims=True)

        def fbody(j, m):
            dc = dist_ref[...]
            sel = jnp.where(dc == m, ii4, iiN)
            idxj = jnp.min(jnp.min(jnp.min(
                sel, axis=2, keepdims=True), axis=1, keepdims=True),
                axis=3, keepdims=True)
            idx_ref[0] = jnp.where(ik == j, idxj[:, :, 0, 0], idx_ref[0])
            dn = jnp.where(ii4 == idxj, _BIG, dc)
            dist_ref[...] = dn
            return jnp.min(_vmin(dn), axis=3, keepdims=True)

        m0 = jnp.min(_vmin(d), axis=3, keepdims=True)
        lax.fori_loop(0, K, fbody, m0)


def _run_knn(lc_xyz, xyz):
    xyzr = xyz.reshape(B, NCH, NEH, NEL, 3).transpose(0, 4, 2, 3, 1)
    xyzr = xyzr.reshape(B * 3, NEH, NEL, NCH)
    return pl.pallas_call(
        _knn_body,
        grid=(B, M // QBLK),
        in_specs=[
            pl.BlockSpec((1, QBLK, 3), lambda b, s: (b, s, 0)),
            pl.BlockSpec((3, NEH, NEL, NCH), lambda b, s: (b, 0, 0, 0)),
        ],
        out_specs=pl.BlockSpec((1, QBLK, K), lambda b, s: (b, s, 0)),
        out_shape=jax.ShapeDtypeStruct((B, M, K), jnp.int32),
        scratch_shapes=[
            pltpu.VMEM(_SH4, jnp.float32),
            pltpu.VMEM((QBLK, TPC * NCH), jnp.float32),
            pltpu.VMEM((QBLK, TPC * NCH), jnp.int32),
        ],
    )(lc_xyz, xyzr)


_NC = 2
_NS = 16
_NW = _NC * _NS
_CHUNK = 128
_LC_PER_W = (B * M) // _NW
_KNN_PER_W = (B * M * K) // _NW
_KNN_CHUNKS = _KNN_PER_W // _CHUNK


def _sc_gather_body(xf_hbm, xyzp_hbm, fps_hbm, knn_hbm,
                    lcx_out, kxyz_out, kx_out,
                    lci_v, lcr_v, ki_v, kxyzr_v, kxr_v, sem1, sem2):
    wid = lax.axis_index("s") * _NC + lax.axis_index("c")
    pltpu.sync_copy(fps_hbm.at[pl.ds(wid * _LC_PER_W, _LC_PER_W)], lci_v)
    pltpu.async_copy(xf_hbm.at[lci_v], lcr_v, sem1).wait()
    pltpu.sync_copy(lcr_v, lcx_out.at[pl.ds(wid * _LC_PER_W, _LC_PER_W)])
    pltpu.sync_copy(knn_hbm.at[pl.ds(wid * _KNN_CHUNKS, _KNN_CHUNKS)], ki_v)

    def chunk(j, _):
        row = ki_v.at[j]
        cp1 = pltpu.async_copy(xyzp_hbm.at[row], kxyzr_v, sem1)
        cp2 = pltpu.async_copy(xf_hbm.at[row], kxr_v, sem2)
        cp1.wait()
        cp2.wait()
        base = wid * _KNN_PER_W + j * _CHUNK
        pltpu.sync_copy(kxyzr_v, kxyz_out.at[pl.ds(base, _CHUNK)])
        pltpu.sync_copy(kxr_v, kx_out.at[pl.ds(base, _CHUNK)])
        return 0

    lax.fori_loop(0, _KNN_CHUNKS, chunk, 0)


@functools.cache
def _make_sc_gather():
    return functools.partial(
        pl.kernel,
        mesh=plsc.VectorSubcoreMesh(core_axis_name="c", subcore_axis_name="s",
                                    num_cores=_NC, num_subcores=_NS),
        out_type=[
            jax.ShapeDtypeStruct((B * M, C_FEAT), jnp.float32),
            jax.ShapeDtypeStruct((B * M * K, 16), jnp.float32),
            jax.ShapeDtypeStruct((B * M * K, C_FEAT), jnp.float32),
        ],
        scratch_types=[
            pltpu.VMEM((_LC_PER_W,), jnp.int32),
            pltpu.VMEM((_LC_PER_W, C_FEAT), jnp.float32),
            pltpu.VMEM((_KNN_CHUNKS, _CHUNK), jnp.int32),
            pltpu.VMEM((_CHUNK, 16), jnp.float32),
            pltpu.VMEM((_CHUNK, C_FEAT), jnp.float32),
            pltpu.SemaphoreType.DMA,
            pltpu.SemaphoreType.DMA,
        ],
        compiler_params=pltpu.CompilerParams(use_tc_tiling_on_sc=False),
    )(_sc_gather_body)


def kernel(xyz, x):
    fps_idx, lct = _run_fps(xyz)
    lc_xyz = lct.transpose(1, 2, 0)
    knn_idx = _run_knn(lc_xyz, xyz)

    xf = x.reshape(B * N, C_FEAT)
    xyzp = jnp.pad(xyz, ((0, 0), (0, 0), (0, 13))).reshape(B * N, 16)
    base = jnp.arange(B, dtype=jnp.int32) * N
    fps_flat = (fps_idx + base[:, None]).reshape(-1)
    knn_flat = (knn_idx + base[:, None, None]).reshape(B * M * K // _CHUNK,
                                                       _CHUNK)
    lcx_rows, kxyz_rows, kx_rows = _make_sc_gather()(xf, xyzp, fps_flat,
                                                     knn_flat)

    lc_x = lcx_rows.reshape(B, M, C_FEAT)
    knn_xyz = kxyz_rows.reshape(B, M, K, 16)[..., :3]
    knn_x = kx_rows.reshape(B, M, K, C_FEAT)
    return lc_xyz, lc_x, knn_xyz, knn_x
